# Initial kernel scaffold; baseline (speedup 1.0000x reference)
import jax
import jax.numpy as jnp
from jax.experimental import pallas as pl


def kernel(edge_index, edge_logits, node_feats, W_proj, b_proj, W_ih, W_hh, b_ih, b_hh):
    raise NotImplementedError("write your pallas kernel here")


# speedup vs baseline: 15.4544x; 15.4544x over previous
import functools

import jax
import jax.numpy as jnp
from jax import lax
from jax.experimental import pallas as pl
from jax.experimental.pallas import tpu as pltpu
from jax.experimental.pallas import tpu_sc as plsc

N = 10000
E = 320000
D = 128
H = 128

NC = 2
NS = 16
CH = 80
CPT = E // (NC * NS * CH)
RPT = N // NS
DZ = 624


def _edge_body(src_hbm, dst_hbm, lg_hbm, hv_hbm, acc_hbm, den_hbm,
               idx_s, idx_d, exb, rows, zden, acc_sh, den_sh):
    c = lax.axis_index("c")
    s = lax.axis_index("s")

    zeros16 = jnp.zeros((16,), jnp.float32)

    def _zrow(r, _):
        for j in range(D // 16):
            rows[r, pl.ds(j * 16, 16)] = zeros16
        return 0
    lax.fori_loop(0, CH, _zrow, 0)
    for j in range((DZ + 16) // 16):
        zden[pl.ds(j * 16, 16)] = zeros16

    zb = DZ * s
    for t in range(8):
        pltpu.sync_copy(rows, acc_sh.at[pl.ds(zb + t * CH, CH), :])
    pltpu.sync_copy(zden, den_sh.at[pl.ds(zb, DZ + 16)])

    pltpu.sync_copy(src_hbm.at[c, s], idx_s)
    pltpu.sync_copy(dst_hbm.at[c, s], idx_d)
    pltpu.sync_copy(lg_hbm.at[c, s], exb)

    def _expr(r, _):
        for j in range(CH // 16):
            exb[r, pl.ds(j * 16, 16)] = jnp.exp(exb[r, pl.ds(j * 16, 16)])
        return 0
    lax.fori_loop(0, CPT, _expr, 0)

    plsc.subcore_barrier()

    def _chunk(g, _):
        pltpu.sync_copy(hv_hbm.at[idx_s.at[g]], rows)
        for eb in range(CH // 16):
            wv = exb[g, pl.ds(eb * 16, 16)]
            for k in range(16):
                e = eb * 16 + k
                w = wv[k]
                for j in range(D // 16):
                    rows[e, pl.ds(j * 16, 16)] = rows[e, pl.ds(j * 16, 16)] * w
        pltpu.sync_copy(rows, acc_sh.at[idx_d.at[g]], add=True)
        pltpu.sync_copy(exb.at[g], den_sh.at[idx_d.at[g]], add=True)
        return 0
    lax.fori_loop(0, CPT, _chunk, 0)

    plsc.subcore_barrier()

    rb = RPT * s
    pltpu.sync_copy(acc_sh.at[pl.ds(rb, RPT), :], acc_hbm.at[c, pl.ds(rb, RPT), :])
    pltpu.sync_copy(den_sh.at[pl.ds(DZ * s, DZ)], den_hbm.at[c, pl.ds(DZ * s, DZ)])

    @pl.when(s == NS - 1)
    def _():
        pltpu.sync_copy(den_sh.at[pl.ds(DZ * NS, N - DZ * NS)],
                        den_hbm.at[c, pl.ds(DZ * NS, N - DZ * NS)])


def _edge_pass(src2, dst2, lg2, hv):
    mesh = plsc.VectorSubcoreMesh(core_axis_name="c", subcore_axis_name="s",
                                  num_cores=NC, num_subcores=NS)
    f = pl.kernel(
        _edge_body,
        compiler_params=pltpu.CompilerParams(use_tc_tiling_on_sc=False),
        out_type=(
            jax.ShapeDtypeStruct((NC, N, D), jnp.float32),
            jax.ShapeDtypeStruct((NC, N), jnp.float32),
        ),
        mesh=mesh,
        scratch_types=[
            pltpu.VMEM((CPT, CH), jnp.int32),
            pltpu.VMEM((CPT, CH), jnp.int32),
            pltpu.VMEM((CPT, CH), jnp.float32),
            pltpu.VMEM((CH, D), jnp.float32),
            pltpu.VMEM((DZ + 16,), jnp.float32),
            pltpu.VMEM_SHARED((N, D), jnp.float32),
            pltpu.VMEM_SHARED((N,), jnp.float32),
        ],
    )
    return f(src2, dst2, lg2, hv)


def _proj_body(nf_ref, w_ref, b_ref, out_ref):
    out_ref[...] = jnp.dot(nf_ref[...], w_ref[...],
                           preferred_element_type=jnp.float32,
                           precision=lax.Precision.HIGHEST) + b_ref[...]


def _proj(node_feats, w_t, b):
    R = 2000
    return pl.pallas_call(
        _proj_body,
        grid=(N // R,),
        in_specs=[
            pl.BlockSpec((R, D), lambda i: (i, 0)),
            pl.BlockSpec((D, H), lambda i: (0, 0)),
            pl.BlockSpec((1, H), lambda i: (0, 0)),
        ],
        out_specs=pl.BlockSpec((R, H), lambda i: (i, 0)),
        out_shape=jax.ShapeDtypeStruct((N, H), jnp.float32),
    )(node_feats, w_t, b)


def _gru_body(acc_ref, den_ref, nf_ref, wih_ref, whh_ref, bih_ref, bhh_ref,
              out_ref):
    csum = acc_ref[0] + acc_ref[1]
    den = den_ref[0, 0, 0] + den_ref[1, 0, 0]
    den = jnp.where(den > 0.0, den, 1.0)
    ctx = csum / den[:, None]
    ctx = jnp.where(ctx > 0.0, ctx, jnp.exp(jnp.minimum(ctx, 0.0)) - 1.0)
    gi = jnp.dot(ctx, wih_ref[...], preferred_element_type=jnp.float32,
                 precision=lax.Precision.HIGHEST) + bih_ref[...]
    gh = jnp.dot(nf_ref[...], whh_ref[...], preferred_element_type=jnp.float32,
                 precision=lax.Precision.HIGHEST) + bhh_ref[...]
    r = jax.nn.sigmoid(gi[:, :D] + gh[:, :D])
    z = jax.nn.sigmoid(gi[:, D:2 * D] + gh[:, D:2 * D])
    n = jnp.tanh(gi[:, 2 * D:] + r * gh[:, 2 * D:])
    h = (1.0 - z) * n + z * nf_ref[...]
    out_ref[...] = jnp.maximum(h, 0.0)


def _gru(acc, den, node_feats, wih_t, whh_t, bih, bhh):
    R = 2000
    return pl.pallas_call(
        _gru_body,
        grid=(N // R,),
        in_specs=[
            pl.BlockSpec((NC, R, D), lambda i: (0, i, 0)),
            pl.BlockSpec((NC, 1, 1, R), lambda i: (0, i, 0, 0)),
            pl.BlockSpec((R, D), lambda i: (i, 0)),
            pl.BlockSpec((H, 3 * D), lambda i: (0, 0)),
            pl.BlockSpec((D, 3 * D), lambda i: (0, 0)),
            pl.BlockSpec((1, 3 * D), lambda i: (0, 0)),
            pl.BlockSpec((1, 3 * D), lambda i: (0, 0)),
        ],
        out_specs=pl.BlockSpec((R, D), lambda i: (i, 0)),
        out_shape=jax.ShapeDtypeStruct((N, D), jnp.float32),
    )(acc, den.reshape(NC, N // R, 1, R), node_feats, wih_t, whh_t, bih, bhh)


def kernel(edge_index, edge_logits, node_feats, W_proj, b_proj, W_ih, W_hh, b_ih, b_hh):
    src2 = edge_index[0].reshape(NC, NS, CPT, CH)
    dst2 = edge_index[1].reshape(NC, NS, CPT, CH)
    lg2 = edge_logits.reshape(NC, NS, CPT, CH)
    hv = _proj(node_feats, W_proj.T, b_proj.reshape(1, H))
    acc, den = _edge_pass(src2, dst2, lg2, hv)
    return _gru(acc, den, node_feats, W_ih.T, W_hh.T,
                b_ih.reshape(1, 3 * D), b_hh.reshape(1, 3 * D))


# speedup vs baseline: 16.2023x; 1.0484x over previous
import functools

import jax
import jax.numpy as jnp
from jax import lax
from jax.experimental import pallas as pl
from jax.experimental.pallas import tpu as pltpu
from jax.experimental.pallas import tpu_sc as plsc

N = 10000
E = 320000
D = 128
H = 128

NC = 2
NS = 16
CH = 80
CPT = E // (NC * NS * CH)
RPT = N // NS
DZ = 624


def _edge_body(edata_hbm, lg_hbm, hv_hbm, acc_hbm, den_hbm,
               ebuf0, ebuf1, ebuf2, ebuf3, lgs, exb0, exb1, rows0, rows1, zden,
               acc_sh, den_sh, esem0, esem1, esem2, esem3, gsem0, gsem1):
    c = lax.axis_index("c")
    s = lax.axis_index("s")

    ebufs = [ebuf0, ebuf1, ebuf2, ebuf3]
    esems = [esem0, esem1, esem2, esem3]
    rbufs = [rows0, rows1]
    gsems = [gsem0, gsem1]
    exbs = [exb0, exb1]

    zeros16 = jnp.zeros((16,), jnp.float32)

    def _zrow(r, _):
        for j in range(D // 16):
            rows0[r, pl.ds(j * 16, 16)] = zeros16
        return 0
    lax.fori_loop(0, CH, _zrow, 0)
    for j in range((DZ + 16) // 16):
        zden[pl.ds(j * 16, 16)] = zeros16

    zb = DZ * s
    for t in range(8):
        pltpu.sync_copy(rows0, acc_sh.at[pl.ds(zb + t * CH, CH), :])
    pltpu.sync_copy(zden, den_sh.at[pl.ds(zb, DZ + 16)])

    pltpu.sync_copy(lg_hbm.at[c, s], lgs)

    plsc.subcore_barrier()

    def _fire_edata(g, be):
        pltpu.async_copy(edata_hbm.at[c, s, g], ebufs[be], esems[be])

    def _wait_edata(g, be):
        pltpu.make_async_copy(edata_hbm.at[c, s, g], ebufs[be], esems[be]).wait()

    def _fire_gather(be, br):
        pltpu.async_copy(hv_hbm.at[ebufs[be].at[0]], rbufs[br], gsems[br])

    def _wait_gather(be, br):
        pltpu.make_async_copy(hv_hbm.at[ebufs[be].at[0]], rbufs[br],
                              gsems[br]).wait()

    def _compute_chunk(g, be, br):
        ebuf, rows, exbuf = ebufs[be], rbufs[br], exbs[br]
        for eb in range(CH // 16):
            lgv = lgs[g, pl.ds(eb * 16, 16)]
            ex = jnp.exp(lgv)
            exbuf[pl.ds(eb * 16, 16)] = ex
            for k in range(16):
                w = ex[k]
                e = eb * 16 + k
                for j in range(D // 16):
                    rows[e, pl.ds(j * 16, 16)] = rows[e, pl.ds(j * 16, 16)] * w
        pltpu.sync_copy(rows, acc_sh.at[ebuf.at[1]], add=True)
        pltpu.sync_copy(exbuf, den_sh.at[ebuf.at[1]], add=True)

    _fire_edata(0, 0)
    _fire_edata(1, 1)
    _fire_edata(2, 2)
    _wait_edata(0, 0)
    _fire_gather(0, 0)

    def _quad(q, _):
        g = q * 4
        for u in range(4):
            gu = g + u
            be, br = u % 4, u % 2
            ben, brn = (u + 1) % 4, (u + 1) % 2
            bef = (u + 3) % 4
            _wait_edata(gu + 1, ben)
            _fire_gather(ben, brn)

            @pl.when(gu + 3 < CPT)
            def _():
                _fire_edata(gu + 3, bef)

            _wait_gather(be, br)
            _compute_chunk(gu, be, br)
        return 0
    lax.fori_loop(0, (CPT - 1) // 4, _quad, 0)

    _wait_gather(0, 0)
    _compute_chunk(CPT - 1, 0, 0)

    plsc.subcore_barrier()

    rb = RPT * s
    pltpu.sync_copy(acc_sh.at[pl.ds(rb, RPT), :], acc_hbm.at[c, pl.ds(rb, RPT), :])
    pltpu.sync_copy(den_sh.at[pl.ds(DZ * s, DZ)], den_hbm.at[c, pl.ds(DZ * s, DZ)])

    @pl.when(s == NS - 1)
    def _():
        pltpu.sync_copy(den_sh.at[pl.ds(DZ * NS, N - DZ * NS)],
                        den_hbm.at[c, pl.ds(DZ * NS, N - DZ * NS)])


def _edge_pass(edata, lg2, hv):
    mesh = plsc.VectorSubcoreMesh(core_axis_name="c", subcore_axis_name="s",
                                  num_cores=NC, num_subcores=NS)
    f = pl.kernel(
        _edge_body,
        compiler_params=pltpu.CompilerParams(use_tc_tiling_on_sc=False),
        out_type=(
            jax.ShapeDtypeStruct((NC, N, D), jnp.float32),
            jax.ShapeDtypeStruct((NC, N), jnp.float32),
        ),
        mesh=mesh,
        scratch_types=[
            pltpu.VMEM((2, CH), jnp.int32),
            pltpu.VMEM((2, CH), jnp.int32),
            pltpu.VMEM((2, CH), jnp.int32),
            pltpu.VMEM((2, CH), jnp.int32),
            pltpu.VMEM((CPT, CH), jnp.float32),
            pltpu.VMEM((CH,), jnp.float32),
            pltpu.VMEM((CH,), jnp.float32),
            pltpu.VMEM((CH, D), jnp.float32),
            pltpu.VMEM((CH, D), jnp.float32),
            pltpu.VMEM((DZ + 16,), jnp.float32),
            pltpu.VMEM_SHARED((N, D), jnp.float32),
            pltpu.VMEM_SHARED((N,), jnp.float32),
            pltpu.SemaphoreType.DMA,
            pltpu.SemaphoreType.DMA,
            pltpu.SemaphoreType.DMA,
            pltpu.SemaphoreType.DMA,
            pltpu.SemaphoreType.DMA,
            pltpu.SemaphoreType.DMA,
        ],
    )
    return f(edata, lg2, hv)


def _proj_body(nf_ref, w_ref, b_ref, out_ref):
    out_ref[...] = jnp.dot(nf_ref[...], w_ref[...],
                           preferred_element_type=jnp.float32,
                           precision=lax.Precision.HIGHEST) + b_ref[...]


def _proj(node_feats, w_t, b):
    R = 2000
    return pl.pallas_call(
        _proj_body,
        grid=(N // R,),
        in_specs=[
            pl.BlockSpec((R, D), lambda i: (i, 0)),
            pl.BlockSpec((D, H), lambda i: (0, 0)),
            pl.BlockSpec((1, H), lambda i: (0, 0)),
        ],
        out_specs=pl.BlockSpec((R, H), lambda i: (i, 0)),
        out_shape=jax.ShapeDtypeStruct((N, H), jnp.float32),
    )(node_feats, w_t, b)


def _gru_body(acc_ref, den_ref, nf_ref, wih_ref, whh_ref, bih_ref, bhh_ref,
              out_ref):
    csum = acc_ref[0] + acc_ref[1]
    den = den_ref[0, 0, 0] + den_ref[1, 0, 0]
    den = jnp.where(den > 0.0, den, 1.0)
    ctx = csum / den[:, None]
    ctx = jnp.where(ctx > 0.0, ctx, jnp.exp(jnp.minimum(ctx, 0.0)) - 1.0)
    gi = jnp.dot(ctx, wih_ref[...], preferred_element_type=jnp.float32,
                 precision=lax.Precision.HIGHEST) + bih_ref[...]
    gh = jnp.dot(nf_ref[...], whh_ref[...], preferred_element_type=jnp.float32,
                 precision=lax.Precision.HIGHEST) + bhh_ref[...]
    r = jax.nn.sigmoid(gi[:, :D] + gh[:, :D])
    z = jax.nn.sigmoid(gi[:, D:2 * D] + gh[:, D:2 * D])
    n = jnp.tanh(gi[:, 2 * D:] + r * gh[:, 2 * D:])
    h = (1.0 - z) * n + z * nf_ref[...]
    out_ref[...] = jnp.maximum(h, 0.0)


def _gru(acc, den, node_feats, wih_t, whh_t, bih, bhh):
    R = 2000
    return pl.pallas_call(
        _gru_body,
        grid=(N // R,),
        in_specs=[
            pl.BlockSpec((NC, R, D), lambda i: (0, i, 0)),
            pl.BlockSpec((NC, 1, 1, R), lambda i: (0, i, 0, 0)),
            pl.BlockSpec((R, D), lambda i: (i, 0)),
            pl.BlockSpec((H, 3 * D), lambda i: (0, 0)),
            pl.BlockSpec((D, 3 * D), lambda i: (0, 0)),
            pl.BlockSpec((1, 3 * D), lambda i: (0, 0)),
            pl.BlockSpec((1, 3 * D), lambda i: (0, 0)),
        ],
        out_specs=pl.BlockSpec((R, D), lambda i: (i, 0)),
        out_shape=jax.ShapeDtypeStruct((N, D), jnp.float32),
    )(acc, den.reshape(NC, N // R, 1, R), node_feats, wih_t, whh_t, bih, bhh)


def kernel(edge_index, edge_logits, node_feats, W_proj, b_proj, W_ih, W_hh, b_ih, b_hh):
    src_i = edge_index[0].reshape(NC, NS, CPT, 1, CH)
    dst_i = edge_index[1].reshape(NC, NS, CPT, 1, CH)
    edata = jnp.concatenate([src_i, dst_i], axis=3)
    lg2 = edge_logits.reshape(NC, NS, CPT, CH)
    hv = _proj(node_feats, W_proj.T, b_proj.reshape(1, H))
    acc, den = _edge_pass(edata, lg2, hv)
    return _gru(acc, den, node_feats, W_ih.T, W_hh.T,
                b_ih.reshape(1, 3 * D), b_hh.reshape(1, 3 * D))


# speedup vs baseline: 16.8986x; 1.0430x over previous
import functools

import jax
import jax.numpy as jnp
from jax import lax
from jax.experimental import pallas as pl
from jax.experimental.pallas import tpu as pltpu
from jax.experimental.pallas import tpu_sc as plsc

N = 10000
E = 320000
D = 128
H = 128

NC = 2
NS = 16
CH = 80
CPT = E // (NC * NS * CH)
RPT = N // NS
DZ = 624


def _edge_body(edata_hbm, lg_hbm, hv_hbm, acc_hbm, den_hbm,
               ebuf0, ebuf1, ebuf2, lgs, exb0, exb1, exb2,
               rows0, rows1, rows2, zden, acc_sh, den_sh,
               esem0, esem1, esem2, gsem0, gsem1, gsem2,
               ssem0, ssem1, ssem2):
    c = lax.axis_index("c")
    s = lax.axis_index("s")

    ebufs = [ebuf0, ebuf1, ebuf2]
    esems = [esem0, esem1, esem2]
    rbufs = [rows0, rows1, rows2]
    gsems = [gsem0, gsem1, gsem2]
    ssems = [ssem0, ssem1, ssem2]
    exbs = [exb0, exb1, exb2]

    zeros16 = jnp.zeros((16,), jnp.float32)

    def _zrow(r, _):
        for j in range(D // 16):
            rows0[r, pl.ds(j * 16, 16)] = zeros16
        return 0
    lax.fori_loop(0, CH, _zrow, 0)
    for j in range((DZ + 16) // 16):
        zden[pl.ds(j * 16, 16)] = zeros16

    zb = DZ * s
    for t in range(8):
        pltpu.sync_copy(rows0, acc_sh.at[pl.ds(zb + t * CH, CH), :])
    pltpu.sync_copy(zden, den_sh.at[pl.ds(zb, DZ + 16)])

    pltpu.sync_copy(lg_hbm.at[c, s], lgs)

    plsc.subcore_barrier()

    def _fire_edata(g, be):
        pltpu.async_copy(edata_hbm.at[c, s, g], ebufs[be], esems[be])

    def _wait_edata(g, be):
        pltpu.make_async_copy(edata_hbm.at[c, s, g], ebufs[be], esems[be]).wait()

    def _fire_gather(be):
        pltpu.async_copy(hv_hbm.at[ebufs[be].at[0]], rbufs[be], gsems[be])

    def _wait_gather(be):
        pltpu.make_async_copy(hv_hbm.at[ebufs[be].at[0]], rbufs[be],
                              gsems[be]).wait()

    def _fire_scatter(be):
        pltpu.async_copy(rbufs[be], acc_sh.at[ebufs[be].at[1]], ssems[be],
                         add=True)

    def _wait_scatter(be):
        pltpu.make_async_copy(rbufs[be], acc_sh.at[ebufs[be].at[1]],
                              ssems[be]).wait()

    def _scale_chunk(g, be):
        rows, exbuf = rbufs[be], exbs[be]
        for eb in range(CH // 16):
            lgv = lgs[g, pl.ds(eb * 16, 16)]
            ex = jnp.exp(lgv)
            exbuf[pl.ds(eb * 16, 16)] = ex
            for k in range(16):
                w = ex[k]
                e = eb * 16 + k
                for j in range(D // 16):
                    rows[e, pl.ds(j * 16, 16)] = rows[e, pl.ds(j * 16, 16)] * w

    def _block(gu, b, last=False):
        bn = (b + 1) % 3
        bp = (b + 2) % 3
        if not last:
            _wait_edata(gu + 1, bn)
            _fire_gather(bn)
        _wait_gather(b)
        _scale_chunk(gu, b)

        @pl.when(gu >= 1)
        def _():
            _wait_scatter(bp)

        @pl.when(gu + 2 < CPT)
        def _():
            _fire_edata(gu + 2, bp)

        _fire_scatter(b)
        pltpu.sync_copy(exbs[b], den_sh.at[ebufs[b].at[1]], add=True)

    _fire_edata(0, 0)
    _fire_edata(1, 1)
    _wait_edata(0, 0)
    _fire_gather(0)

    def _tri(q, _):
        g3 = q * 3
        for u in range(3):
            _block(g3 + u, u)
        return 0
    lax.fori_loop(0, (CPT - 2) // 3, _tri, 0)

    _block(CPT - 2, 0)
    _block(CPT - 1, 1, last=True)
    _wait_scatter(1)

    plsc.subcore_barrier()

    rb = RPT * s
    pltpu.sync_copy(acc_sh.at[pl.ds(rb, RPT), :], acc_hbm.at[c, pl.ds(rb, RPT), :])
    pltpu.sync_copy(den_sh.at[pl.ds(DZ * s, DZ)], den_hbm.at[c, pl.ds(DZ * s, DZ)])

    @pl.when(s == NS - 1)
    def _():
        pltpu.sync_copy(den_sh.at[pl.ds(DZ * NS, N - DZ * NS)],
                        den_hbm.at[c, pl.ds(DZ * NS, N - DZ * NS)])


def _edge_pass(edata, lg2, hv):
    mesh = plsc.VectorSubcoreMesh(core_axis_name="c", subcore_axis_name="s",
                                  num_cores=NC, num_subcores=NS)
    f = pl.kernel(
        _edge_body,
        compiler_params=pltpu.CompilerParams(use_tc_tiling_on_sc=False),
        out_type=(
            jax.ShapeDtypeStruct((NC, N, D), jnp.float32),
            jax.ShapeDtypeStruct((NC, N), jnp.float32),
        ),
        mesh=mesh,
        scratch_types=[
            pltpu.VMEM((2, CH), jnp.int32),
            pltpu.VMEM((2, CH), jnp.int32),
            pltpu.VMEM((2, CH), jnp.int32),
            pltpu.VMEM((CPT, CH), jnp.float32),
            pltpu.VMEM((CH,), jnp.float32),
            pltpu.VMEM((CH,), jnp.float32),
            pltpu.VMEM((CH,), jnp.float32),
            pltpu.VMEM((CH, D), jnp.float32),
            pltpu.VMEM((CH, D), jnp.float32),
            pltpu.VMEM((CH, D), jnp.float32),
            pltpu.VMEM((DZ + 16,), jnp.float32),
            pltpu.VMEM_SHARED((N, D), jnp.float32),
            pltpu.VMEM_SHARED((N,), jnp.float32),
        ] + [pltpu.SemaphoreType.DMA] * 9,
    )
    return f(edata, lg2, hv)


def _proj_body(nf_ref, w_ref, b_ref, out_ref):
    out_ref[...] = jnp.dot(nf_ref[...], w_ref[...],
                           preferred_element_type=jnp.float32,
                           precision=lax.Precision.HIGHEST) + b_ref[...]


def _proj(node_feats, w_t, b):
    R = 2000
    return pl.pallas_call(
        _proj_body,
        grid=(N // R,),
        in_specs=[
            pl.BlockSpec((R, D), lambda i: (i, 0)),
            pl.BlockSpec((D, H), lambda i: (0, 0)),
            pl.BlockSpec((1, H), lambda i: (0, 0)),
        ],
        out_specs=pl.BlockSpec((R, H), lambda i: (i, 0)),
        out_shape=jax.ShapeDtypeStruct((N, H), jnp.float32),
    )(node_feats, w_t, b)


def _gru_body(acc_ref, den_ref, nf_ref, wih_ref, whh_ref, bih_ref, bhh_ref,
              out_ref):
    csum = acc_ref[0] + acc_ref[1]
    den = den_ref[0, 0, 0] + den_ref[1, 0, 0]
    den = jnp.where(den > 0.0, den, 1.0)
    ctx = csum / den[:, None]
    ctx = jnp.where(ctx > 0.0, ctx, jnp.exp(jnp.minimum(ctx, 0.0)) - 1.0)
    gi = jnp.dot(ctx, wih_ref[...], preferred_element_type=jnp.float32,
                 precision=lax.Precision.HIGHEST) + bih_ref[...]
    gh = jnp.dot(nf_ref[...], whh_ref[...], preferred_element_type=jnp.float32,
                 precision=lax.Precision.HIGHEST) + bhh_ref[...]
    r = jax.nn.sigmoid(gi[:, :D] + gh[:, :D])
    z = jax.nn.sigmoid(gi[:, D:2 * D] + gh[:, D:2 * D])
    n = jnp.tanh(gi[:, 2 * D:] + r * gh[:, 2 * D:])
    h = (1.0 - z) * n + z * nf_ref[...]
    out_ref[...] = jnp.maximum(h, 0.0)


def _gru(acc, den, node_feats, wih_t, whh_t, bih, bhh):
    R = 2000
    return pl.pallas_call(
        _gru_body,
        grid=(N // R,),
        in_specs=[
            pl.BlockSpec((NC, R, D), lambda i: (0, i, 0)),
            pl.BlockSpec((NC, 1, 1, R), lambda i: (0, i, 0, 0)),
            pl.BlockSpec((R, D), lambda i: (i, 0)),
            pl.BlockSpec((H, 3 * D), lambda i: (0, 0)),
            pl.BlockSpec((D, 3 * D), lambda i: (0, 0)),
            pl.BlockSpec((1, 3 * D), lambda i: (0, 0)),
            pl.BlockSpec((1, 3 * D), lambda i: (0, 0)),
        ],
        out_specs=pl.BlockSpec((R, D), lambda i: (i, 0)),
        out_shape=jax.ShapeDtypeStruct((N, D), jnp.float32),
    )(acc, den.reshape(NC, N // R, 1, R), node_feats, wih_t, whh_t, bih, bhh)


def kernel(edge_index, edge_logits, node_feats, W_proj, b_proj, W_ih, W_hh, b_ih, b_hh):
    src_i = edge_index[0].reshape(NC, NS, CPT, 1, CH)
    dst_i = edge_index[1].reshape(NC, NS, CPT, 1, CH)
    edata = jnp.concatenate([src_i, dst_i], axis=3)
    lg2 = edge_logits.reshape(NC, NS, CPT, CH)
    hv = _proj(node_feats, W_proj.T, b_proj.reshape(1, H))
    acc, den = _edge_pass(edata, lg2, hv)
    return _gru(acc, den, node_feats, W_ih.T, W_hh.T,
                b_ih.reshape(1, 3 * D), b_hh.reshape(1, 3 * D))
